# Initial kernel scaffold; baseline (speedup 1.0000x reference)
import jax
import jax.numpy as jnp
from jax.experimental import pallas as pl


def kernel(sampling_map, refined_response_maps, selected_regions):
    raise NotImplementedError("write your pallas kernel here")


# speedup vs baseline: 3.0044x; 3.0044x over previous
import functools

import jax
import jax.numpy as jnp
from jax.experimental import pallas as pl
from jax.experimental.pallas import tpu as pltpu

_GRID_SIZE = 4
_WINDOW_GRID_SIZE = 3


def _fuse_body(sel_ref, samp_ref, ref_ref, out_ref, *, H, W, band_h, top_k):
    b = pl.program_id(0)
    ty = pl.program_id(1)
    grid_h = H // _GRID_SIZE
    grid_w = W // _GRID_SIZE

    yy = jax.lax.broadcasted_iota(jnp.int32, (band_h, W), 0) + ty * band_h
    xx = jax.lax.broadcasted_iota(jnp.int32, (band_h, W), 1)

    fused = samp_ref[0]
    for k in range(top_k):
        rs = sel_ref[b * (2 * top_k) + 2 * k]
        cs = sel_ref[b * (2 * top_k) + 2 * k + 1]
        y0 = jnp.maximum(rs * grid_h, 0)
        y1 = jnp.minimum(y0 + _WINDOW_GRID_SIZE * grid_h, H)
        x0 = jnp.maximum(cs * grid_w, 0)
        x1 = jnp.minimum(x0 + _WINDOW_GRID_SIZE * grid_w, W)
        mask = (yy >= y0) & (yy < y1) & (xx >= x0) & (xx < x1)
        fused = jnp.where(mask[None], ref_ref[0, k], fused)
    out_ref[0] = jax.nn.sigmoid(fused)


def kernel(sampling_map, refined_response_maps, selected_regions):
    B, C, H, W = sampling_map.shape
    top_k = refined_response_maps.shape[1]
    band_h = H // _GRID_SIZE
    c_blk = 32
    n_cb = C // c_blk

    sel_flat = selected_regions.reshape(-1).astype(jnp.int32)

    body = functools.partial(_fuse_body, H=H, W=W, band_h=band_h, top_k=top_k)
    out = pl.pallas_call(
        body,
        grid=(B, _GRID_SIZE, n_cb),
        in_specs=[
            pl.BlockSpec(memory_space=pltpu.SMEM),
            pl.BlockSpec((1, c_blk, band_h, W), lambda b, ty, cb: (b, cb, ty, 0)),
            pl.BlockSpec((1, top_k, c_blk, band_h, W),
                         lambda b, ty, cb: (b, 0, cb, ty, 0)),
        ],
        out_specs=pl.BlockSpec((1, c_blk, band_h, W),
                               lambda b, ty, cb: (b, cb, ty, 0)),
        out_shape=jax.ShapeDtypeStruct((B, C, H, W), jnp.float32),
    )(sel_flat, sampling_map, refined_response_maps)
    return out


# speedup vs baseline: 5.6920x; 1.8946x over previous
import functools

import jax
import jax.numpy as jnp
from jax.experimental import pallas as pl
from jax.experimental.pallas import tpu as pltpu

_GRID_SIZE = 4
_WINDOW_GRID_SIZE = 3


def _fuse_body(sel_ref, samp_hbm, ref_hbm, out_ref,
               buf_a, buf_b, sem_a, sem_b,
               *, C, W, band_h, tw, top_k, n_steps):
    i = pl.program_id(0) * _GRID_SIZE + pl.program_id(1)
    slot = jax.lax.rem(i, 2)

    def winners(step):
        b = step // _GRID_SIZE
        ty = jax.lax.rem(step, _GRID_SIZE)
        ws = []
        for tx in range(_GRID_SIZE):
            w = jnp.int32(-1)
            for k in range(top_k):
                rs = sel_ref[b * 2 * top_k + 2 * k]
                cs = sel_ref[b * 2 * top_k + 2 * k + 1]
                cover = ((ty >= rs) & (ty < rs + _WINDOW_GRID_SIZE)
                         & (tx >= cs) & (tx < cs + _WINDOW_GRID_SIZE))
                w = jnp.where(cover, jnp.int32(k), w)
            ws.append(w)
        return b, ws

    def src_pair(ws):
        src_a = ws[1]
        src_b = jnp.where(ws[0] != ws[1], ws[0], ws[-1])
        return src_a, src_b

    def start_band(step, slot_):
        b, ws = winners(step)
        ty = jax.lax.rem(step, _GRID_SIZE)
        ys = ty * band_h
        src_a, src_b = src_pair(ws)
        for src, buf, sem in ((src_a, buf_a, sem_a), (src_b, buf_b, sem_b)):
            @pl.when(src >= 0)
            def _():
                pltpu.make_async_copy(
                    ref_hbm.at[b, jnp.maximum(src, 0), :, pl.ds(ys, band_h), :],
                    buf.at[slot_], sem.at[slot_]).start()

            @pl.when(src < 0)
            def _():
                pltpu.make_async_copy(
                    samp_hbm.at[b, :, pl.ds(ys, band_h), :],
                    buf.at[slot_], sem.at[slot_]).start()

    @pl.when(i == 0)
    def _():
        start_band(i, slot)

    @pl.when(i + 1 < n_steps)
    def _():
        start_band(i + 1, 1 - slot)

    for buf, sem in ((buf_a, sem_a), (buf_b, sem_b)):
        pltpu.make_async_copy(
            samp_hbm.at[0, :, pl.ds(0, band_h), :],
            buf.at[slot], sem.at[slot]).wait()

    _, ws = winners(i)
    src_a, _ = src_pair(ws)
    xx = jax.lax.broadcasted_iota(jnp.int32, (band_h, W), 1)
    use_a = jnp.zeros((band_h, W), dtype=jnp.bool_)
    for tx in range(_GRID_SIZE):
        m = (ws[tx] == src_a) & (xx >= tx * tw) & (xx < (tx + 1) * tw)
        use_a = use_a | m
    fused = jnp.where(use_a[None], buf_a[slot], buf_b[slot])
    out_ref[0] = jax.nn.sigmoid(fused)


def kernel(sampling_map, refined_response_maps, selected_regions):
    B, C, H, W = sampling_map.shape
    top_k = refined_response_maps.shape[1]
    band_h = H // _GRID_SIZE
    tw = W // _GRID_SIZE
    n_steps = B * _GRID_SIZE

    sel_flat = selected_regions.reshape(-1).astype(jnp.int32)

    body = functools.partial(
        _fuse_body, C=C, W=W, band_h=band_h, tw=tw, top_k=top_k,
        n_steps=n_steps)
    out = pl.pallas_call(
        body,
        grid=(B, _GRID_SIZE),
        in_specs=[
            pl.BlockSpec(memory_space=pltpu.SMEM),
            pl.BlockSpec(memory_space=pl.ANY),
            pl.BlockSpec(memory_space=pl.ANY),
        ],
        out_specs=pl.BlockSpec((1, C, band_h, W), lambda b, ty: (b, 0, ty, 0)),
        out_shape=jax.ShapeDtypeStruct((B, C, H, W), jnp.float32),
        scratch_shapes=[
            pltpu.VMEM((2, C, band_h, W), jnp.float32),
            pltpu.VMEM((2, C, band_h, W), jnp.float32),
            pltpu.SemaphoreType.DMA((2,)),
            pltpu.SemaphoreType.DMA((2,)),
        ],
    )(sel_flat, sampling_map, refined_response_maps)
    return out


# speedup vs baseline: 5.8769x; 1.0325x over previous
import functools

import jax
import jax.numpy as jnp
from jax.experimental import pallas as pl
from jax.experimental.pallas import tpu as pltpu

_GRID_SIZE = 4
_WINDOW_GRID_SIZE = 3


def _fuse_body(sel_ref, samp_hbm, ref_hbm, out_ref,
               buf_a, buf_bl, buf_bf, sem_a, sem_bl, sem_bf,
               *, C, W, band_h, tw, xsplit, top_k, n_steps):
    i = pl.program_id(0) * _GRID_SIZE + pl.program_id(1)
    slot = jax.lax.rem(i, 2)

    def winners(step):
        b = step // _GRID_SIZE
        ty = jax.lax.rem(step, _GRID_SIZE)
        ws = []
        for tx in range(_GRID_SIZE):
            w = jnp.int32(-1)
            for k in range(top_k):
                rs = sel_ref[b * 2 * top_k + 2 * k]
                cs = sel_ref[b * 2 * top_k + 2 * k + 1]
                cover = ((ty >= rs) & (ty < rs + _WINDOW_GRID_SIZE)
                         & (tx >= cs) & (tx < cs + _WINDOW_GRID_SIZE))
                w = jnp.where(cover, jnp.int32(k), w)
            ws.append(w)
        return b, ws

    def start_band(step, slot_):
        b, ws = winners(step)
        ty = jax.lax.rem(step, _GRID_SIZE)
        ys = ty * band_h
        src_a = ws[1]
        left_diff = ws[0] != src_a
        right_diff = ws[-1] != src_a
        src_b = jnp.where(left_diff, ws[0], ws[-1])

        @pl.when(src_a >= 0)
        def _():
            pltpu.make_async_copy(
                ref_hbm.at[b, jnp.maximum(src_a, 0), :, pl.ds(ys, band_h), :],
                buf_a.at[slot_], sem_a.at[slot_]).start()

        @pl.when(src_a < 0)
        def _():
            pltpu.make_async_copy(
                samp_hbm.at[b, :, pl.ds(ys, band_h), :],
                buf_a.at[slot_], sem_a.at[slot_]).start()

        for diff, x0, xn, buf, sem in (
                (left_diff, 0, xsplit, buf_bl, sem_bl),
                (right_diff, 0, W, buf_bf, sem_bf)):
            @pl.when(diff & (src_b >= 0))
            def _():
                pltpu.make_async_copy(
                    ref_hbm.at[b, jnp.maximum(src_b, 0), :,
                               pl.ds(ys, band_h), pl.ds(x0, xn)],
                    buf.at[slot_], sem.at[slot_]).start()

            @pl.when(diff & (src_b < 0))
            def _():
                pltpu.make_async_copy(
                    samp_hbm.at[b, :, pl.ds(ys, band_h), pl.ds(x0, xn)],
                    buf.at[slot_], sem.at[slot_]).start()

    @pl.when(i == 0)
    def _():
        start_band(i, slot)

    @pl.when(i + 1 < n_steps)
    def _():
        start_band(i + 1, 1 - slot)

    _, ws = winners(i)
    src_a = ws[1]
    left_diff = ws[0] != src_a
    right_diff = ws[-1] != src_a

    pltpu.make_async_copy(
        samp_hbm.at[0, :, pl.ds(0, band_h), :],
        buf_a.at[slot], sem_a.at[slot]).wait()

    @pl.when(left_diff)
    def _():
        pltpu.make_async_copy(
            samp_hbm.at[0, :, pl.ds(0, band_h), pl.ds(0, xsplit)],
            buf_bl.at[slot], sem_bl.at[slot]).wait()

    @pl.when(right_diff)
    def _():
        pltpu.make_async_copy(
            samp_hbm.at[0, :, pl.ds(0, band_h), :],
            buf_bf.at[slot], sem_bf.at[slot]).wait()

    xx_l = jax.lax.broadcasted_iota(jnp.int32, (band_h, xsplit), 1)
    mask_l = left_diff & (xx_l < tw)
    left = jnp.where(mask_l[None], buf_bl[slot], buf_a[slot, :, :, 0:xsplit])
    out_ref[0, :, :, 0:xsplit] = jax.nn.sigmoid(left)

    xx_r = jax.lax.broadcasted_iota(jnp.int32, (band_h, W - xsplit), 1) + xsplit
    mask_r = right_diff & (xx_r >= (_GRID_SIZE - 1) * tw)
    right = jnp.where(mask_r[None], buf_bf[slot, :, :, xsplit:W],
                      buf_a[slot, :, :, xsplit:W])
    out_ref[0, :, :, xsplit:W] = jax.nn.sigmoid(right)


def kernel(sampling_map, refined_response_maps, selected_regions):
    B, C, H, W = sampling_map.shape
    top_k = refined_response_maps.shape[1]
    band_h = H // _GRID_SIZE
    tw = W // _GRID_SIZE
    n_steps = B * _GRID_SIZE
    xsplit = 128 if (tw <= 128 <= (_GRID_SIZE - 1) * tw) else 2 * tw

    sel_flat = selected_regions.reshape(-1).astype(jnp.int32)

    body = functools.partial(
        _fuse_body, C=C, W=W, band_h=band_h, tw=tw, xsplit=xsplit,
        top_k=top_k, n_steps=n_steps)
    out = pl.pallas_call(
        body,
        grid=(B, _GRID_SIZE),
        in_specs=[
            pl.BlockSpec(memory_space=pltpu.SMEM),
            pl.BlockSpec(memory_space=pl.ANY),
            pl.BlockSpec(memory_space=pl.ANY),
        ],
        out_specs=pl.BlockSpec((1, C, band_h, W), lambda b, ty: (b, 0, ty, 0)),
        out_shape=jax.ShapeDtypeStruct((B, C, H, W), jnp.float32),
        scratch_shapes=[
            pltpu.VMEM((2, C, band_h, W), jnp.float32),
            pltpu.VMEM((2, C, band_h, xsplit), jnp.float32),
            pltpu.VMEM((2, C, band_h, W), jnp.float32),
            pltpu.SemaphoreType.DMA((2,)),
            pltpu.SemaphoreType.DMA((2,)),
            pltpu.SemaphoreType.DMA((2,)),
        ],
    )(sel_flat, sampling_map, refined_response_maps)
    return out
